# baseline (speedup 1.0000x reference)
import functools

import jax
import jax.numpy as jnp
from jax import lax
from jax.experimental import pallas as pl
from jax.experimental.pallas import tpu as pltpu
from jax.experimental.pallas import tpu_sc as plsc

NUM_HEADS = 16
MAX_DISTANCE = 4
NUM_BUCKETS = 2 * MAX_DISTANCE + 1
SIDE = 32
LEN = SIDE * SIDE
EXT = 2 * LEN
TAB_PAD = 96
ROWS_PER_TILE = SIDE // 2
CHUNKS = EXT // 16


def _bucket(m):
    w = ((m + SIDE // 2) % SIDE) - SIDE // 2
    return jnp.clip(w, -MAX_DISTANCE, MAX_DISTANCE) + MAX_DISTANCE


def _index_matrix():
    p = jnp.arange(EXT)
    xi = jnp.arange(SIDE)
    ypart = NUM_BUCKETS * _bucket((-(p // SIDE)) % SIDE)
    xpart = _bucket((xi[:, None] - p[None, :]) % SIDE)
    return (ypart[None, :] + xpart).astype(jnp.int32)


def _sc_body(tab_hbm, idx_hbm, out_hbm, tabrow_v, idxs_v, band_v, sem):
    par = lax.axis_index("c")
    h = lax.axis_index("s")

    pltpu.sync_copy(tab_hbm.at[h], tabrow_v)
    pltpu.sync_copy(idx_hbm.at[pl.ds(par * ROWS_PER_TILE, ROWS_PER_TILE), :],
                    idxs_v)

    def chunk_body(c, carry):
        col = c * 16
        for r in range(ROWS_PER_TILE):
            iv = idxs_v[r, pl.ds(col, 16)]
            band_v[r, pl.ds(col, 16)] = plsc.load_gather(tabrow_v, [iv])
        return carry

    lax.fori_loop(0, CHUNKS, chunk_body, 0)

    copies = []
    for yi in range(SIDE):
        src = band_v.at[:, pl.ds(LEN - SIDE * yi, LEN)]
        dst = out_hbm.at[0, h, pl.ds(yi * SIDE + par * ROWS_PER_TILE,
                                     ROWS_PER_TILE), :]
        cp = pltpu.make_async_copy(src, dst, sem)
        cp.start()
        copies.append(cp)
    for cp in copies:
        cp.wait()


@jax.jit
def _bias_sc(tab, idxmat):
    mesh = plsc.VectorSubcoreMesh(core_axis_name="c", subcore_axis_name="s")
    run = pl.kernel(
        _sc_body,
        mesh=mesh,
        out_type=jax.ShapeDtypeStruct((1, NUM_HEADS, LEN, LEN), jnp.float32),
        scratch_types=[
            pltpu.VMEM((TAB_PAD,), jnp.float32),
            pltpu.VMEM((ROWS_PER_TILE, EXT), jnp.int32),
            pltpu.VMEM((ROWS_PER_TILE, EXT), jnp.float32),
            pltpu.SemaphoreType.DMA,
        ],
        compiler_params=pltpu.CompilerParams(use_tc_tiling_on_sc=False,
                                             needs_layout_passes=False),
    )
    return run(tab, idxmat)


def kernel(height, width, rel_embedding):
    tab = jnp.pad(rel_embedding.reshape(NUM_HEADS, NUM_BUCKETS * NUM_BUCKETS),
                  ((0, 0), (0, TAB_PAD - NUM_BUCKETS * NUM_BUCKETS)))
    return _bias_sc(tab, _index_matrix())


# speedup vs baseline: 2.2670x; 2.2670x over previous
import functools

import jax
import jax.numpy as jnp
from jax import lax
from jax.experimental import pallas as pl
from jax.experimental.pallas import tpu as pltpu
from jax.experimental.pallas import tpu_sc as plsc

NUM_HEADS = 16
MAX_DISTANCE = 4
NUM_BUCKETS = 2 * MAX_DISTANCE + 1
SIDE = 32
LEN = SIDE * SIDE
EXT = 2 * LEN
TAB_PAD = 96
ROWS_PER_TILE = SIDE // 2
CHUNKS = EXT // 16


def _bucket(m):
    w = ((m + SIDE // 2) % SIDE) - SIDE // 2
    return jnp.clip(w, -MAX_DISTANCE, MAX_DISTANCE) + MAX_DISTANCE


def _index_matrix():
    p = jnp.arange(EXT)
    xi = jnp.arange(SIDE)
    ypart = NUM_BUCKETS * _bucket((-(p // SIDE)) % SIDE)
    xpart = _bucket((xi[:, None] - p[None, :]) % SIDE)
    return (ypart[None, :] + xpart).astype(jnp.int32)


def _sc_body(tab_hbm, idx_hbm, out_hbm, tabrow_v, idxs_v, band_v, sem):
    par = lax.axis_index("c")
    h = lax.axis_index("s")

    pltpu.sync_copy(tab_hbm.at[h], tabrow_v)
    pltpu.sync_copy(idx_hbm.at[pl.ds(par * ROWS_PER_TILE, ROWS_PER_TILE), :],
                    idxs_v)

    def chunk_body(c, carry):
        col = c * 16
        for r in range(ROWS_PER_TILE):
            iv = idxs_v[r, pl.ds(col, 16)]
            band_v[r, pl.ds(col, 16)] = plsc.load_gather(tabrow_v, [iv])
        return carry

    lax.fori_loop(0, CHUNKS, chunk_body, 0)

    def dma_body(i, carry):
        yi = i // 16
        ipar = (i // 8) % 2
        j = i % 8
        off = pl.multiple_of(LEN - SIDE * yi + 128 * j, 32)
        src = band_v.at[pl.ds(8 * ipar, 8), pl.ds(off, 128)]
        dst = out_hbm.at[0, h, 4 * yi + 2 * par + ipar, j]
        pltpu.make_async_copy(src, dst, sem).start()
        return carry

    lax.fori_loop(0, 2 * SIDE * 8, dma_body, 0)

    def drain_body(i, carry):
        pltpu.make_async_copy(band_v.at[pl.ds(0, 8), pl.ds(0, 128)],
                              out_hbm.at[0, h, 0, 0], sem).wait()
        return carry

    lax.fori_loop(0, 2 * SIDE * 8, drain_body, 0)


@jax.jit
def _bias_sc(tab, idxmat):
    mesh = plsc.VectorSubcoreMesh(core_axis_name="c", subcore_axis_name="s")
    run = pl.kernel(
        _sc_body,
        mesh=mesh,
        out_type=jax.ShapeDtypeStruct((1, NUM_HEADS, LEN // 8, 8, 8, 128),
                                      jnp.float32),
        scratch_types=[
            pltpu.VMEM((TAB_PAD,), jnp.float32),
            pltpu.VMEM((ROWS_PER_TILE, EXT), jnp.int32),
            pltpu.VMEM((ROWS_PER_TILE, EXT), jnp.float32),
            pltpu.SemaphoreType.DMA,
        ],
        compiler_params=pltpu.CompilerParams(use_tc_tiling_on_sc=False,
                                             needs_layout_passes=False),
    )
    return run(tab, idxmat)


def kernel(height, width, rel_embedding):
    tab = jnp.pad(rel_embedding.reshape(NUM_HEADS, NUM_BUCKETS * NUM_BUCKETS),
                  ((0, 0), (0, TAB_PAD - NUM_BUCKETS * NUM_BUCKETS)))
    out6 = _bias_sc(tab, _index_matrix())
    return out6.transpose(0, 1, 2, 4, 3, 5).reshape(1, NUM_HEADS, LEN, LEN)
